# baseline (speedup 1.0000x reference)
import numpy as np

import jax
import jax.numpy as jnp
from jax.experimental import pallas as pl
from jax.experimental.pallas import tpu as pltpu

B = 4
IMG = 224
P = 16
NE = 256
NH = 8
HS = NE // NH
NL = 4
E = 8
TOPK = 2
FD = 256
T = (IMG // P) ** 2
FF = 4 * NE
TP = 208
R = B * TP
SCALE = NE ** -0.5

_NEG = -1e30


def _ln_rows(v, g, b):
    m = jnp.mean(v, axis=1, keepdims=True)
    d = v - m
    var = jnp.mean(d * d, axis=1, keepdims=True)
    return d / jnp.sqrt(var + 1e-5) * g + b


def _dot_t(a, bmat, prec=None):
    return jax.lax.dot_general(a, bmat, (((1,), (1,)), ((), ())),
                               preferred_element_type=jnp.float32,
                               precision=prec)


def _dot(a, bmat, prec=None):
    return jax.lax.dot_general(a, bmat, (((1,), (0,)), ((), ())),
                               preferred_element_type=jnp.float32,
                               precision=prec)


_HI = jax.lax.Precision.HIGHEST


def _bf(v):
    return v.astype(jnp.bfloat16)


def _model_kernel(xp_ref, convw_ref, ebias_ref, wqkv_ref, projw_ref,
                  rtnz_ref, miscl_ref, b1_ref, b2_ref, w1_ref, w2_ref,
                  nrm_ref, sel_ref, headw_ref, fmisc_ref, out_ref,
                  t_ref, hfb_ref, gate_ref):
    s = pl.program_id(0)
    ei = jax.lax.rem(s, E)
    misc = miscl_ref[0]

    @pl.when(s == 0)
    def _embed():
        t_ref[...] = _dot_t(xp_ref[...], convw_ref[...]) + ebias_ref[...]

    @pl.when(ei == 0)
    def _dense_stage():
        t = t_ref[...]

        h = _ln_rows(t, misc[0:1, :], misc[1:2, :])
        qkv = _dot_t(_bf(h), _bf(wqkv_ref[0]))

        lane = jax.lax.broadcasted_iota(jnp.int32, (TP, NE), 1)
        rowi = jax.lax.broadcasted_iota(jnp.int32, (TP, TP), 0)
        coli = jax.lax.broadcasted_iota(jnp.int32, (TP, TP), 1)
        causal = coli <= rowi

        att_rows = []
        for b in range(B):
            qb = qkv[b * TP:(b + 1) * TP, 0:NE]
            kb = _bf(qkv[b * TP:(b + 1) * TP, NE:2 * NE])
            vb = qkv[b * TP:(b + 1) * TP, 2 * NE:3 * NE]
            att_b = jnp.zeros((TP, NE), jnp.float32)
            for hd in range(NH):
                mh = (lane // HS) == hd
                qh = _bf(jnp.where(mh, qb, 0.0))
                sc = _dot_t(qh, kb) * SCALE
                sc = jnp.where(causal, sc, _NEG)
                smax = jnp.max(sc, axis=1, keepdims=True)
                p = jnp.exp(sc - smax)
                p = p / jnp.sum(p, axis=1, keepdims=True)
                vh = _bf(jnp.where(mh, vb, 0.0))
                att_b = att_b + _dot(_bf(p), vh)
            att_rows.append(att_b)
        att = jnp.concatenate(att_rows, axis=0)

        t = t + _dot_t(_bf(att), _bf(projw_ref[0])) + misc[4:5, :]

        h2 = _ln_rows(t, misc[2:3, :], misc[3:4, :])
        hfb = _bf(h2)
        hfb_ref[...] = hfb
        lg = _dot_t(hfb, _bf(rtnz_ref[0])) + misc[5:6, :]
        logits = lg[:, 0:128]
        nlog = lg[:, 128:256]
        sp = jnp.maximum(nlog, 0.0) + jnp.log1p(jnp.exp(-jnp.abs(nlog)))
        noisy = logits + nrm_ref[0] * sp

        lane8 = jax.lax.broadcasted_iota(jnp.int32, (R, 128), 1)
        valid = lane8 < E
        nz = jnp.where(valid, noisy, _NEG)
        m1 = jnp.max(nz, axis=1, keepdims=True)
        i1 = jnp.min(jnp.where((nz == m1) & valid, lane8, 127), axis=1,
                     keepdims=True)
        oh1 = lane8 == i1
        nz2 = jnp.where(oh1, _NEG, nz)
        m2 = jnp.max(nz2, axis=1, keepdims=True)
        i2 = jnp.min(jnp.where((nz2 == m2) & valid, lane8, 127), axis=1,
                     keepdims=True)
        oh2 = lane8 == i2
        e2 = jnp.exp(m2 - m1)
        g1 = 1.0 / (1.0 + e2)
        g2 = e2 * g1
        gate_ref[...] = (g1 * oh1.astype(jnp.float32)
                         + g2 * oh2.astype(jnp.float32))
        t_ref[...] = t

    hfb = hfb_ref[...]
    lane8 = jax.lax.broadcasted_iota(jnp.int32, (R, 128), 1)
    a = _dot_t(hfb, _bf(w1_ref[0, 0])) + b1_ref[0, 0]
    a = jnp.maximum(a, 0.0)
    o = _dot_t(_bf(a), _bf(w2_ref[0, 0])) + b2_ref[0, 0]
    ge = jnp.sum(jnp.where(lane8 == ei, gate_ref[...], 0.0), axis=1,
                 keepdims=True)
    t_ref[...] = t_ref[...] + ge * o

    @pl.when(s == NL * E - 1)
    def _finish():
        t = t_ref[...]
        fm = fmisc_ref[...]
        y = _ln_rows(t, fm[0:1, :], fm[1:2, :])
        mb = _dot(sel_ref[...], y, _HI)
        out_ref[...] = _dot_t(_bf(mb), headw_ref[...]) + fm[2:3, :]


def _build_call():
    const = lambda nd: (lambda i: (0,) * nd)
    perl = lambda nd: (lambda i: (i // E,) + (0,) * (nd - 1))
    perle = lambda nd: (lambda i: (i // E, jax.lax.rem(i, E))
                        + (0,) * (nd - 2))

    in_specs = [
        pl.BlockSpec((R, 768), const(2)),
        pl.BlockSpec((NE, 768), const(2)),
        pl.BlockSpec((R, NE), const(2)),
        pl.BlockSpec((1, 3 * NE, NE), perl(3)),
        pl.BlockSpec((1, NE, NE), perl(3)),
        pl.BlockSpec((1, NE, NE), perl(3)),
        pl.BlockSpec((1, 8, NE), perl(3)),
        pl.BlockSpec((1, 1, 1, FF), perle(4)),
        pl.BlockSpec((1, 1, 1, NE), perle(4)),
        pl.BlockSpec((1, 1, FF, NE), perle(4)),
        pl.BlockSpec((1, 1, NE, FF), perle(4)),
        pl.BlockSpec((1, R, 128), perl(3)),
        pl.BlockSpec((8, R), const(2)),
        pl.BlockSpec((FD, NE), const(2)),
        pl.BlockSpec((8, NE), const(2)),
    ]
    out_spec = pl.BlockSpec((8, FD), const(2))
    out_shape = jax.ShapeDtypeStruct((8, FD), jnp.float32)

    return pl.pallas_call(
        _model_kernel,
        grid=(NL * E,),
        in_specs=in_specs,
        out_specs=out_spec,
        out_shape=out_shape,
        scratch_shapes=[pltpu.VMEM((R, NE), jnp.float32),
                        pltpu.VMEM((R, NE), jnp.bfloat16),
                        pltpu.VMEM((R, 128), jnp.float32)],
    )


_CALL = _build_call()

_SEL = np.zeros((8, R), np.float32)
for _b in range(B):
    _SEL[_b, _b * TP:_b * TP + T] = 1.0 / T


@jax.jit
def _run(xp, convw, ebias, wqkvs, projws, rtnzs, miscls, b1s, b2s, w1s,
         w2s, nrms, sel, headw, fmisc):
    out = _CALL(xp, convw, ebias, wqkvs, projws, rtnzs, miscls, b1s, b2s,
                w1s, w2s, nrms, sel, headw, fmisc)
    return out[:B]


def kernel(x, params):
    f32 = jnp.float32
    bf16 = jnp.bfloat16

    xp = x.reshape(B, 3, IMG // P, P, IMG // P, P)
    xp = xp.transpose(0, 2, 4, 1, 3, 5).reshape(B, T, 3 * P * P)
    xp = jnp.pad(xp, ((0, 0), (0, TP - T), (0, 0))).reshape(R, 3 * P * P)

    convw = params["conv_w"].reshape(NE, 3 * P * P)
    eb = params["pos"][0] + params["conv_b"]
    ebias = jnp.tile(jnp.pad(eb, ((0, TP - T), (0, 0))), (B, 1))

    Ls = params["layers"]
    wqkvs, projws, rtnzs, miscls, b1s, b2s, w1s, w2s = ([] for _ in range(8))
    for L in Ls:
        wqkvs.append(jnp.concatenate(
            [L["wq"].reshape(NE, NE), L["wk"].reshape(NE, NE),
             L["wv"].reshape(NE, NE)], axis=0))
        projws.append(L["proj_w"])
        rtnzs.append(jnp.zeros((NE, NE), f32).at[0:E].set(L["rt_w"])
                     .at[128:128 + E].set(L["nz_w"]))
        miscls.append(jnp.stack(
            [L["ln1_g"], L["ln1_b"], L["ln2_g"], L["ln2_b"], L["proj_b"],
             jnp.zeros((NE,), f32).at[0:E].set(L["rt_b"])
             .at[128:128 + E].set(L["nz_b"]),
             jnp.zeros((NE,), f32), jnp.zeros((NE,), f32)]))
        b1s.append(L["e_b1"].reshape(E, 1, FF))
        b2s.append(L["e_b2"].reshape(E, 1, NE))
        w1s.append(L["e_w1"])
        w2s.append(L["e_w2"])

    nkey = jax.random.key(42)
    nrms = []
    for li in range(NL):
        nr = jax.random.normal(jax.random.fold_in(nkey, li), (B, T, E), f32)
        nr = jnp.pad(nr, ((0, 0), (0, TP - T), (0, 128 - E)))
        nrms.append(nr.reshape(R, 128))

    sel = jnp.asarray(_SEL)
    fmisc = jnp.stack([params["lnf_g"], params["lnf_b"], params["head_b"],
                       jnp.zeros((NE,), f32), jnp.zeros((NE,), f32),
                       jnp.zeros((NE,), f32), jnp.zeros((NE,), f32),
                       jnp.zeros((NE,), f32)])

    return _run(xp.astype(bf16), convw.astype(bf16), ebias,
                jnp.stack(wqkvs), jnp.stack(projws), jnp.stack(rtnzs),
                jnp.stack(miscls), jnp.stack(b1s), jnp.stack(b2s),
                jnp.stack(w1s), jnp.stack(w2s), jnp.stack(nrms), sel,
                params["head_w"].astype(bf16), fmisc)


# speedup vs baseline: 1.0440x; 1.0440x over previous
import numpy as np

import jax
import jax.numpy as jnp
from jax.experimental import pallas as pl
from jax.experimental.pallas import tpu as pltpu

B = 4
IMG = 224
P = 16
NE = 256
NH = 8
HS = NE // NH
NL = 4
E = 8
TOPK = 2
FD = 256
T = (IMG // P) ** 2
FF = 4 * NE
TP = 208
R = B * TP
SCALE = NE ** -0.5

_NEG = -1e30


def _ln_rows(v, g, b):
    m = jnp.mean(v, axis=1, keepdims=True)
    d = v - m
    var = jnp.mean(d * d, axis=1, keepdims=True)
    return d / jnp.sqrt(var + 1e-5) * g + b


def _dot_t(a, bmat, prec=None):
    return jax.lax.dot_general(a, bmat, (((1,), (1,)), ((), ())),
                               preferred_element_type=jnp.float32,
                               precision=prec)


def _dot(a, bmat, prec=None):
    return jax.lax.dot_general(a, bmat, (((1,), (0,)), ((), ())),
                               preferred_element_type=jnp.float32,
                               precision=prec)


_HI = jax.lax.Precision.HIGHEST


def _bf(v):
    return v.astype(jnp.bfloat16)


def _layer_kernel(first, last, *refs):
    if first:
        (xp_ref, convw_ref, ebias_ref, wqkv_ref, projw_ref, rtnz_ref,
         miscl_ref, b1_ref, b2_ref, w1_ref, w2_ref, nrm_ref) = refs[:12]
        refs = refs[12:]
    else:
        (tin_ref, wqkv_ref, projw_ref, rtnz_ref, miscl_ref, b1_ref,
         b2_ref, w1_ref, w2_ref, nrm_ref) = refs[:10]
        refs = refs[10:]
    if last:
        sel_ref, headw_ref, fmisc_ref, out_ref = refs[:4]
        refs = refs[4:]
    else:
        out_ref = refs[0]
        refs = refs[1:]
    t_ref, hfb_ref, gate_ref = refs

    ei = pl.program_id(0)
    misc = miscl_ref[...]

    @pl.when(ei == 0)
    def _dense_stage():
        if first:
            t = _dot_t(xp_ref[...], convw_ref[...]) + ebias_ref[...]
        else:
            t = tin_ref[...]

        h = _ln_rows(t, misc[0:1, :], misc[1:2, :])
        qkv = _dot_t(_bf(h), _bf(wqkv_ref[...]))

        lane = jax.lax.broadcasted_iota(jnp.int32, (TP, NE), 1)
        rowi = jax.lax.broadcasted_iota(jnp.int32, (TP, TP), 0)
        coli = jax.lax.broadcasted_iota(jnp.int32, (TP, TP), 1)
        causal = coli <= rowi

        att_rows = []
        for b in range(B):
            qb = qkv[b * TP:(b + 1) * TP, 0:NE]
            kb = _bf(qkv[b * TP:(b + 1) * TP, NE:2 * NE])
            vb = qkv[b * TP:(b + 1) * TP, 2 * NE:3 * NE]
            att_b = jnp.zeros((TP, NE), jnp.float32)
            for hd in range(NH):
                mh = (lane // HS) == hd
                qh = _bf(jnp.where(mh, qb, 0.0))
                s = _dot_t(qh, kb) * SCALE
                s = jnp.where(causal, s, _NEG)
                smax = jnp.max(s, axis=1, keepdims=True)
                p = jnp.exp(s - smax)
                p = p / jnp.sum(p, axis=1, keepdims=True)
                vh = _bf(jnp.where(mh, vb, 0.0))
                att_b = att_b + _dot(_bf(p), vh)
            att_rows.append(att_b)
        att = jnp.concatenate(att_rows, axis=0)

        t = t + _dot_t(_bf(att), _bf(projw_ref[...])) + misc[4:5, :]

        h2 = _ln_rows(t, misc[2:3, :], misc[3:4, :])
        hfb = _bf(h2)
        hfb_ref[...] = hfb
        lg = _dot_t(hfb, _bf(rtnz_ref[...])) + misc[5:6, :]
        logits = lg[:, 0:128]
        nlog = lg[:, 128:256]
        sp = jnp.maximum(nlog, 0.0) + jnp.log1p(jnp.exp(-jnp.abs(nlog)))
        noisy = logits + nrm_ref[...] * sp

        lane8 = jax.lax.broadcasted_iota(jnp.int32, (R, 128), 1)
        valid = lane8 < E
        nz = jnp.where(valid, noisy, _NEG)
        m1 = jnp.max(nz, axis=1, keepdims=True)
        i1 = jnp.min(jnp.where((nz == m1) & valid, lane8, 127), axis=1,
                     keepdims=True)
        oh1 = lane8 == i1
        nz2 = jnp.where(oh1, _NEG, nz)
        m2 = jnp.max(nz2, axis=1, keepdims=True)
        i2 = jnp.min(jnp.where((nz2 == m2) & valid, lane8, 127), axis=1,
                     keepdims=True)
        oh2 = lane8 == i2
        e2 = jnp.exp(m2 - m1)
        g1 = 1.0 / (1.0 + e2)
        g2 = e2 * g1
        gate_ref[...] = (g1 * oh1.astype(jnp.float32)
                         + g2 * oh2.astype(jnp.float32))
        t_ref[...] = t

    hfb = hfb_ref[...]
    lane8 = jax.lax.broadcasted_iota(jnp.int32, (R, 128), 1)
    a = _dot_t(hfb, w1_ref[0]) + b1_ref[0]
    a = jnp.maximum(a, 0.0)
    o = _dot_t(_bf(a), w2_ref[0]) + b2_ref[0]
    ge = jnp.sum(jnp.where(lane8 == ei, gate_ref[...], 0.0), axis=1,
                 keepdims=True)
    t_ref[...] = t_ref[...] + ge * o

    @pl.when(ei == E - 1)
    def _finish():
        t = t_ref[...]
        if last:
            fm = fmisc_ref[...]
            y = _ln_rows(t, fm[0:1, :], fm[1:2, :])
            mb = _dot(sel_ref[...], y, _HI)
            out_ref[...] = (_dot_t(_bf(mb), headw_ref[...])
                            + fm[2:3, :])
        else:
            out_ref[...] = t


def _build_call(first, last):
    const = lambda nd: (lambda i: (0,) * nd)
    pere = lambda nd: (lambda i: (i,) + (0,) * (nd - 1))

    in_specs = []
    if first:
        in_specs += [
            pl.BlockSpec((R, 768), const(2)),
            pl.BlockSpec((NE, 768), const(2)),
            pl.BlockSpec((R, NE), const(2)),
        ]
    else:
        in_specs += [pl.BlockSpec((R, NE), const(2))]
    in_specs += [
        pl.BlockSpec((3 * NE, NE), const(2)),
        pl.BlockSpec((NE, NE), const(2)),
        pl.BlockSpec((NE, NE), const(2)),
        pl.BlockSpec((8, NE), const(2)),
        pl.BlockSpec((1, 1, FF), pere(3)),
        pl.BlockSpec((1, 1, NE), pere(3)),
        pl.BlockSpec((1, FF, NE), pere(3)),
        pl.BlockSpec((1, NE, FF), pere(3)),
        pl.BlockSpec((R, 128), const(2)),
    ]
    if last:
        in_specs += [
            pl.BlockSpec((8, R), const(2)),
            pl.BlockSpec((FD, NE), const(2)),
            pl.BlockSpec((8, NE), const(2)),
        ]
        out_spec = pl.BlockSpec((8, FD), const(2))
        out_shape = jax.ShapeDtypeStruct((8, FD), jnp.float32)
    else:
        out_spec = pl.BlockSpec((R, NE), const(2))
        out_shape = jax.ShapeDtypeStruct((R, NE), jnp.float32)

    def body(*refs):
        _layer_kernel(first, last, *refs)

    return pl.pallas_call(
        body,
        grid=(E,),
        in_specs=in_specs,
        out_specs=out_spec,
        out_shape=out_shape,
        scratch_shapes=[pltpu.VMEM((R, NE), jnp.float32),
                        pltpu.VMEM((R, NE), jnp.bfloat16),
                        pltpu.VMEM((R, 128), jnp.float32)],
    )


_CALL_FIRST = _build_call(True, False)
_CALL_MID = _build_call(False, False)
_CALL_LAST = _build_call(False, True)

_SEL = np.zeros((8, R), np.float32)
for _b in range(B):
    _SEL[_b, _b * TP:_b * TP + T] = 1.0 / T


@jax.jit
def _run(xp, convw, ebias, wqkvs, projws, rtnzs, miscls, b1s, b2s, w1s,
         w2s, nrms, sel, headw, fmisc):
    t = None
    for li in range(NL):
        common = (wqkvs[li], projws[li], rtnzs[li], miscls[li], b1s[li],
                  b2s[li], w1s[li], w2s[li], nrms[li])
        if li == 0:
            t = _CALL_FIRST(xp, convw, ebias, *common)
        elif li < NL - 1:
            t = _CALL_MID(t, *common)
        else:
            out = _CALL_LAST(t, *common, sel, headw, fmisc)
    return out[:B]


def kernel(x, params):
    f32 = jnp.float32
    bf16 = jnp.bfloat16

    xp = x.reshape(B, 3, IMG // P, P, IMG // P, P)
    xp = xp.transpose(0, 2, 4, 1, 3, 5).reshape(B, T, 3 * P * P)
    xp = jnp.pad(xp, ((0, 0), (0, TP - T), (0, 0))).reshape(R, 3 * P * P)

    convw = params["conv_w"].reshape(NE, 3 * P * P)
    eb = params["pos"][0] + params["conv_b"]
    ebias = jnp.tile(jnp.pad(eb, ((0, TP - T), (0, 0))), (B, 1))

    Ls = params["layers"]
    wqkvs, projws, rtnzs, miscls, b1s, b2s, w1s, w2s = ([] for _ in range(8))
    for L in Ls:
        wqkvs.append(jnp.concatenate(
            [L["wq"].reshape(NE, NE), L["wk"].reshape(NE, NE),
             L["wv"].reshape(NE, NE)], axis=0))
        projws.append(L["proj_w"])
        rtnzs.append(jnp.zeros((NE, NE), f32).at[0:E].set(L["rt_w"])
                     .at[128:128 + E].set(L["nz_w"]))
        miscls.append(jnp.stack(
            [L["ln1_g"], L["ln1_b"], L["ln2_g"], L["ln2_b"], L["proj_b"],
             jnp.zeros((NE,), f32).at[0:E].set(L["rt_b"])
             .at[128:128 + E].set(L["nz_b"]),
             jnp.zeros((NE,), f32), jnp.zeros((NE,), f32)]))
        b1s.append(L["e_b1"].reshape(E, 1, FF))
        b2s.append(L["e_b2"].reshape(E, 1, NE))
        w1s.append(L["e_w1"].astype(bf16))
        w2s.append(L["e_w2"].astype(bf16))

    nkey = jax.random.key(42)
    nrms = []
    for li in range(NL):
        nr = jax.random.normal(jax.random.fold_in(nkey, li), (B, T, E), f32)
        nr = jnp.pad(nr, ((0, 0), (0, TP - T), (0, 128 - E)))
        nrms.append(nr.reshape(R, 128))

    sel = jnp.asarray(_SEL)
    fmisc = jnp.stack([params["lnf_g"], params["lnf_b"], params["head_b"],
                       jnp.zeros((NE,), f32), jnp.zeros((NE,), f32),
                       jnp.zeros((NE,), f32), jnp.zeros((NE,), f32),
                       jnp.zeros((NE,), f32)])

    return _run(xp.astype(bf16), convw.astype(bf16), ebias, wqkvs, projws,
                rtnzs, miscls, b1s, b2s, w1s, w2s, nrms, sel,
                params["head_w"].astype(bf16), fmisc)


# speedup vs baseline: 1.5947x; 1.5275x over previous
import numpy as np

import jax
import jax.numpy as jnp
from jax.experimental import pallas as pl
from jax.experimental.pallas import tpu as pltpu

B = 4
IMG = 224
P = 16
NE = 256
NH = 8
HS = NE // NH
NL = 4
E = 8
TOPK = 2
FD = 256
T = (IMG // P) ** 2
FF = 4 * NE
TP = 208
R = B * TP
SCALE = NE ** -0.5

_NEG = -1e30


def _ln_rows(v, g, b):
    m = jnp.mean(v, axis=1, keepdims=True)
    d = v - m
    var = jnp.mean(d * d, axis=1, keepdims=True)
    return d / jnp.sqrt(var + 1e-5) * g + b


def _dot_t(a, bmat, prec=None):
    return jax.lax.dot_general(a, bmat, (((1,), (1,)), ((), ())),
                               preferred_element_type=jnp.float32,
                               precision=prec)


def _dot(a, bmat, prec=None):
    return jax.lax.dot_general(a, bmat, (((1,), (0,)), ((), ())),
                               preferred_element_type=jnp.float32,
                               precision=prec)


_HI = jax.lax.Precision.HIGHEST


def _bf(v):
    return v.astype(jnp.bfloat16)


def _layer_kernel(first, last, *refs):
    if first:
        xp_ref, convw_ref, ebias_ref = refs[:3]
        refs = refs[3:]
    else:
        tin_ref = refs[0]
        refs = refs[1:]
    (wq_ref, wk_ref, wv_ref, projw_ref, rtw_ref, nzw_ref, ln1g_ref,
     ln1b_ref, ln2g_ref, ln2b_ref, projb_ref, rtb_ref, nzb_ref,
     b1_ref, b2_ref, w1_ref, w2_ref, nrm_ref) = refs[:18]
    refs = refs[18:]
    if last:
        sel_ref, headw_ref, lnfg_ref, lnfb_ref, headb_ref, out_ref = refs[:6]
        refs = refs[6:]
    else:
        out_ref = refs[0]
        refs = refs[1:]
    t_ref, hfb_ref, gate_ref = refs

    ei = pl.program_id(0)

    @pl.when(ei == 0)
    def _dense_stage():
        if first:
            t = _dot_t(xp_ref[...], convw_ref[...]) + ebias_ref[...]
        else:
            t = tin_ref[...]

        h = _bf(_ln_rows(t, ln1g_ref[...], ln1b_ref[...]))
        q = _dot_t(h, _bf(wq_ref[...]))
        k = _dot_t(h, _bf(wk_ref[...]))
        v = _dot_t(h, _bf(wv_ref[...]))

        lane = jax.lax.broadcasted_iota(jnp.int32, (TP, NE), 1)
        rowi = jax.lax.broadcasted_iota(jnp.int32, (TP, TP), 0)
        coli = jax.lax.broadcasted_iota(jnp.int32, (TP, TP), 1)
        causal = coli <= rowi

        att_rows = []
        for b in range(B):
            qb = q[b * TP:(b + 1) * TP, :]
            kb = _bf(k[b * TP:(b + 1) * TP, :])
            vb = v[b * TP:(b + 1) * TP, :]
            att_b = jnp.zeros((TP, NE), jnp.float32)
            for hd in range(NH):
                mh = (lane // HS) == hd
                qh = _bf(jnp.where(mh, qb, 0.0))
                s = _dot_t(qh, kb) * SCALE
                s = jnp.where(causal, s, _NEG)
                smax = jnp.max(s, axis=1, keepdims=True)
                p = jnp.exp(s - smax)
                p = p / jnp.sum(p, axis=1, keepdims=True)
                vh = _bf(jnp.where(mh, vb, 0.0))
                att_b = att_b + _dot(_bf(p), vh)
            att_rows.append(att_b)
        att = jnp.concatenate(att_rows, axis=0)

        t = t + _dot_t(_bf(att), _bf(projw_ref[...])) + projb_ref[...]

        h2 = _ln_rows(t, ln2g_ref[...], ln2b_ref[...])
        hfb = _bf(h2)
        hfb_ref[...] = hfb
        logits = _dot_t(hfb, _bf(rtw_ref[...])) + rtb_ref[...]
        nlog = _dot_t(hfb, _bf(nzw_ref[...])) + nzb_ref[...]
        sp = jnp.maximum(nlog, 0.0) + jnp.log1p(jnp.exp(-jnp.abs(nlog)))
        noisy = logits + nrm_ref[...] * sp

        lane8 = jax.lax.broadcasted_iota(jnp.int32, (R, E), 1)
        m1 = jnp.max(noisy, axis=1, keepdims=True)
        i1 = jnp.min(jnp.where(noisy == m1, lane8, E), axis=1,
                     keepdims=True)
        oh1 = lane8 == i1
        nz2 = jnp.where(oh1, _NEG, noisy)
        m2 = jnp.max(nz2, axis=1, keepdims=True)
        i2 = jnp.min(jnp.where(nz2 == m2, lane8, E), axis=1,
                     keepdims=True)
        oh2 = lane8 == i2
        e2 = jnp.exp(m2 - m1)
        g1 = 1.0 / (1.0 + e2)
        g2 = e2 * g1
        gate_ref[...] = (g1 * oh1.astype(jnp.float32)
                         + g2 * oh2.astype(jnp.float32))
        t_ref[...] = t

    hfb = hfb_ref[...]
    lane8 = jax.lax.broadcasted_iota(jnp.int32, (R, E), 1)
    a = _dot_t(hfb, _bf(w1_ref[0])) + b1_ref[0]
    a = jnp.maximum(a, 0.0)
    o = _dot_t(_bf(a), _bf(w2_ref[0])) + b2_ref[0]
    ge = jnp.sum(jnp.where(lane8 == ei, gate_ref[...], 0.0), axis=1,
                 keepdims=True)
    t_ref[...] = t_ref[...] + ge * o

    @pl.when(ei == E - 1)
    def _finish():
        t = t_ref[...]
        if last:
            y = _ln_rows(t, lnfg_ref[...], lnfb_ref[...])
            mb = _dot(sel_ref[...], y, _HI)
            out_ref[...] = (_dot_t(_bf(mb), headw_ref[...])
                            + headb_ref[...])
        else:
            out_ref[...] = t


def _build_call(first, last):
    const = lambda nd: (lambda i: (0,) * nd)
    pere = lambda nd: (lambda i: (i,) + (0,) * (nd - 1))

    in_specs = []
    if first:
        in_specs += [
            pl.BlockSpec((R, 768), const(2)),
            pl.BlockSpec((NE, 768), const(2)),
            pl.BlockSpec((R, NE), const(2)),
        ]
    else:
        in_specs += [pl.BlockSpec((R, NE), const(2))]
    in_specs += [
        pl.BlockSpec((NE, NE), const(2)),
        pl.BlockSpec((NE, NE), const(2)),
        pl.BlockSpec((NE, NE), const(2)),
        pl.BlockSpec((NE, NE), const(2)),
        pl.BlockSpec((E, NE), const(2)),
        pl.BlockSpec((E, NE), const(2)),
        pl.BlockSpec((1, NE), const(2)),
        pl.BlockSpec((1, NE), const(2)),
        pl.BlockSpec((1, NE), const(2)),
        pl.BlockSpec((1, NE), const(2)),
        pl.BlockSpec((1, NE), const(2)),
        pl.BlockSpec((1, E), const(2)),
        pl.BlockSpec((1, E), const(2)),
        pl.BlockSpec((1, 1, FF), pere(3)),
        pl.BlockSpec((1, 1, NE), pere(3)),
        pl.BlockSpec((1, FF, NE), pere(3)),
        pl.BlockSpec((1, NE, FF), pere(3)),
        pl.BlockSpec((R, E), const(2)),
    ]
    if last:
        in_specs += [
            pl.BlockSpec((8, R), const(2)),
            pl.BlockSpec((FD, NE), const(2)),
            pl.BlockSpec((1, NE), const(2)),
            pl.BlockSpec((1, NE), const(2)),
            pl.BlockSpec((1, NE), const(2)),
        ]
        out_spec = pl.BlockSpec((8, FD), const(2))
        out_shape = jax.ShapeDtypeStruct((8, FD), jnp.float32)
    else:
        out_spec = pl.BlockSpec((R, NE), const(2))
        out_shape = jax.ShapeDtypeStruct((R, NE), jnp.float32)

    def body(*refs):
        _layer_kernel(first, last, *refs)

    return pl.pallas_call(
        body,
        grid=(E,),
        in_specs=in_specs,
        out_specs=out_spec,
        out_shape=out_shape,
        scratch_shapes=[pltpu.VMEM((R, NE), jnp.float32),
                        pltpu.VMEM((R, NE), jnp.bfloat16),
                        pltpu.VMEM((R, E), jnp.float32)],
    )


_CALL_FIRST = _build_call(True, False)
_CALL_MID = _build_call(False, False)
_CALL_LAST = _build_call(False, True)

_SEL = np.zeros((8, R), np.float32)
for _b in range(B):
    _SEL[_b, _b * TP:_b * TP + T] = 1.0 / T

_NRMS = []
_nkey = jax.random.key(42)
for _li in range(NL):
    _nr = jax.random.normal(jax.random.fold_in(_nkey, _li), (B, T, E),
                            jnp.float32)
    _nr = jnp.pad(_nr, ((0, 0), (0, TP - T), (0, 0))).reshape(R, E)
    _NRMS.append(_nr)


@jax.jit
def _run(xp, convw, ebias, lws, sel, headw, lnfg, lnfb, headb, nrms):
    t = None
    for li in range(NL):
        if li == 0:
            t = _CALL_FIRST(xp, convw, ebias, *lws[li], nrms[li])
        elif li < NL - 1:
            t = _CALL_MID(t, *lws[li], nrms[li])
        else:
            out = _CALL_LAST(t, *lws[li], nrms[li], sel, headw, lnfg,
                             lnfb, headb)
    return out[:B]


def kernel(x, params):
    bf16 = jnp.bfloat16

    xp = x.reshape(B, 3, IMG // P, P, IMG // P, P)
    xp = xp.transpose(0, 2, 4, 1, 3, 5).reshape(B, T, 3 * P * P)
    xp = jnp.pad(xp, ((0, 0), (0, TP - T), (0, 0))).reshape(R, 3 * P * P)

    convw = params["conv_w"].reshape(NE, 3 * P * P)
    eb = params["pos"][0] + params["conv_b"]
    ebias = jnp.tile(jnp.pad(eb, ((0, TP - T), (0, 0))), (B, 1))

    lws = []
    for L in params["layers"]:
        lws.append((
            L["wq"].reshape(NE, NE), L["wk"].reshape(NE, NE),
            L["wv"].reshape(NE, NE), L["proj_w"], L["rt_w"], L["nz_w"],
            L["ln1_g"].reshape(1, NE), L["ln1_b"].reshape(1, NE),
            L["ln2_g"].reshape(1, NE), L["ln2_b"].reshape(1, NE),
            L["proj_b"].reshape(1, NE), L["rt_b"].reshape(1, E),
            L["nz_b"].reshape(1, E), L["e_b1"].reshape(E, 1, FF),
            L["e_b2"].reshape(E, 1, NE), L["e_w1"], L["e_w2"],
        ))

    return _run(xp.astype(bf16), convw.astype(bf16), ebias, lws,
                jnp.asarray(_SEL), params["head_w"].astype(bf16),
                params["lnf_g"].reshape(1, NE),
                params["lnf_b"].reshape(1, NE),
                params["head_b"].reshape(1, NE), _NRMS)
